# baseline (speedup 1.0000x reference)
import jax
import jax.numpy as jnp
from jax import lax
from jax.experimental import pallas as pl
from jax.experimental.pallas import tpu as pltpu
from jax.experimental.pallas import tpu_sc as plsc

E = 64
TOPK = 2
DMODEL = 1024
DFF = 512
T = 64
FP8_MAX = 448.0
L = 16
NBUF = 3


def _routing_body(gt_hbm, dense_hbm, order_hbm, nu_hbm,
                  gt_v, dense_v, used_v, ord_v, nu_v):
    c = lax.axis_index("c")
    s = lax.axis_index("s")

    @pl.when(jnp.logical_and(c == 0, s == 0))
    def _work():
        pltpu.sync_copy(gt_hbm, gt_v)
        zf = jnp.zeros((L,), jnp.float32)
        zi = jnp.zeros((L,), jnp.int32)

        def _zero_chunk(j, carry):
            for u in range(16):
                dense_v[pl.ds(L * 16 * j + L * u, L)] = zf
            return carry

        lax.fori_loop(0, E * T // (L * 16), _zero_chunk, 0)
        for k in range(E // L):
            used_v[pl.ds(L * k, L)] = zi
            ord_v[pl.ds(L * k, L)] = zi

        ones = jnp.ones((L,), jnp.int32)
        for g in range(T // L):
            def _step(eb, carry):
                m1, i1, m2, i2 = carry
                for u in range(8):
                    e = eb * 8 + u
                    v = gt_v[e, pl.ds(L * g, L)]
                    e_vec = jnp.full((L,), 1, jnp.int32) * e
                    gt1 = v > m1
                    gt2 = jnp.logical_and(jnp.logical_not(gt1), v > m2)
                    m2n = jnp.where(gt1, m1, jnp.where(gt2, v, m2))
                    i2 = jnp.where(gt1, i1, jnp.where(gt2, e_vec, i2))
                    m2 = m2n
                    i1 = jnp.where(gt1, e_vec, i1)
                    m1 = jnp.where(gt1, v, m1)
                return m1, i1, m2, i2

            init = (jnp.full((L,), -jnp.inf, jnp.float32), zi,
                    jnp.full((L,), -jnp.inf, jnp.float32), zi)
            m1, i1, m2, i2 = lax.fori_loop(0, E // 8, _step, init)
            r = jnp.exp(m2 - m1)
            w1v = 1.0 / (1.0 + r)
            w2v = r / (1.0 + r)
            tvec = (lax.iota(jnp.int32, L) + L * g) * E
            plsc.store_scatter(dense_v, [tvec + i1], w1v)
            plsc.store_scatter(dense_v, [tvec + i2], w2v)
            plsc.store_scatter(used_v, [i1], ones)
            plsc.store_scatter(used_v, [i2], ones)

        total = jnp.int32(0)
        for k in range(E // L):
            mk = used_v[pl.ds(L * k, L)] > 0
            inc = jnp.where(mk, 1, 0)
            pos = plsc.cumsum(inc) - 1 + total
            ids = lax.iota(jnp.int32, L) + L * k
            plsc.store_scatter(ord_v, [pos], ids, mask=mk)
            total = total + jnp.sum(inc)
        nu_v[...] = jnp.full((L,), total, jnp.int32)

        pltpu.sync_copy(dense_v, dense_hbm)
        pltpu.sync_copy(ord_v, order_hbm)
        pltpu.sync_copy(nu_v, nu_hbm)


def _routing(gt):
    f = pl.kernel(
        _routing_body,
        out_type=(
            jax.ShapeDtypeStruct((T * E,), jnp.float32),
            jax.ShapeDtypeStruct((E,), jnp.int32),
            jax.ShapeDtypeStruct((L,), jnp.int32),
        ),
        mesh=plsc.VectorSubcoreMesh(core_axis_name="c", subcore_axis_name="s"),
        compiler_params=pltpu.CompilerParams(needs_layout_passes=False),
        scratch_types=[
            pltpu.VMEM((E, T), jnp.float32),
            pltpu.VMEM((T * E,), jnp.float32),
            pltpu.VMEM((E,), jnp.int32),
            pltpu.VMEM((E,), jnp.int32),
            pltpu.VMEM((L,), jnp.int32),
        ],
    )
    return f(gt)


def _moe_body(order_ref, nu_ref, x_ref, dt_ref, s1_ref, s2_ref,
              a1_ref, a2_ref, w1_hbm, w2_hbm, out_ref,
              w1_buf, w2_buf, sem1, sem2):
    nu = nu_ref[0]

    def _issue(i, b):
        e = order_ref[i]
        pltpu.make_async_copy(w1_hbm.at[e], w1_buf.at[b], sem1.at[b]).start()
        pltpu.make_async_copy(w2_hbm.at[e], w2_buf.at[b], sem2.at[b]).start()

    def _wait1(i, b):
        e = order_ref[i]
        pltpu.make_async_copy(w1_hbm.at[e], w1_buf.at[b], sem1.at[b]).wait()

    def _wait2(i, b):
        e = order_ref[i]
        pltpu.make_async_copy(w2_hbm.at[e], w2_buf.at[b], sem2.at[b]).wait()

    for b in range(NBUF):
        @pl.when(b < nu)
        def _prime(b=b):
            _issue(b, b)

    a1 = a1_ref[0]
    a2 = a2_ref[0]
    xq = jnp.clip(x_ref[...] / a1, -FP8_MAX, FP8_MAX)
    dt = dt_ref[...]
    col = lax.broadcasted_iota(jnp.int32, (E, 1), 0)
    out_ref[...] = jnp.zeros_like(out_ref)

    def _super(i_sup, acc):
        base = i_sup * NBUF
        for b in range(NBUF):
            i = base + b

            @pl.when(i < nu)
            def _one(i=i, b=b):
                e = order_ref[i]
                _wait1(i, b)
                h = lax.dot_general(xq, w1_buf[b], (((1,), (1,)), ((), ())),
                                    precision=lax.Precision.HIGHEST,
                                    preferred_element_type=jnp.float32)
                h = h * (a1 * s1_ref[e])
                gate = h[:, :DFF]
                up = h[:, DFF:]
                act = gate * jax.nn.sigmoid(gate) * up
                aq = jnp.clip(act / a2, -FP8_MAX, FP8_MAX)
                _wait2(i, b)
                oe = lax.dot_general(aq, w2_buf[b], (((1,), (1,)), ((), ())),
                                     precision=lax.Precision.HIGHEST,
                                     preferred_element_type=jnp.float32)
                oe = oe * (a2 * s2_ref[e])
                onehot = jnp.where(col == e, 1.0, 0.0)
                wt_col = lax.dot_general(dt, onehot, (((1,), (0,)), ((), ())),
                                         preferred_element_type=jnp.float32)
                out_ref[...] += wt_col * oe

                j = i + NBUF

                @pl.when(j < nu)
                def _next():
                    _issue(j, b)

        return acc

    n_sup = lax.div(nu + (NBUF - 1), NBUF)
    lax.fori_loop(0, n_sup, _super, 0)


def kernel(x, gating_output, w1_q, w2_q, w1_scale, w2_scale, a1_scale,
           a2_scale):
    dense, order, nu = _routing(gating_output.T)
    dt = dense.reshape(T, E)
    s1 = w1_scale.reshape(E)
    s2 = w2_scale.reshape(E)
    a1 = a1_scale.reshape(1)
    a2 = a2_scale.reshape(1)
    return pl.pallas_call(
        _moe_body,
        in_specs=[
            pl.BlockSpec(memory_space=pltpu.SMEM),
            pl.BlockSpec(memory_space=pltpu.SMEM),
            pl.BlockSpec(memory_space=pltpu.VMEM),
            pl.BlockSpec(memory_space=pltpu.VMEM),
            pl.BlockSpec(memory_space=pltpu.SMEM),
            pl.BlockSpec(memory_space=pltpu.SMEM),
            pl.BlockSpec(memory_space=pltpu.SMEM),
            pl.BlockSpec(memory_space=pltpu.SMEM),
            pl.BlockSpec(memory_space=pltpu.MemorySpace.HBM),
            pl.BlockSpec(memory_space=pltpu.MemorySpace.HBM),
        ],
        out_specs=pl.BlockSpec(memory_space=pltpu.VMEM),
        out_shape=jax.ShapeDtypeStruct((T, DMODEL), jnp.float32),
        scratch_shapes=[
            pltpu.VMEM((NBUF, 2 * DFF, DMODEL), jnp.float32),
            pltpu.VMEM((NBUF, DMODEL, DFF), jnp.float32),
            pltpu.SemaphoreType.DMA((NBUF,)),
            pltpu.SemaphoreType.DMA((NBUF,)),
        ],
        compiler_params=pltpu.CompilerParams(
            vmem_limit_bytes=100 * 1024 * 1024),
    )(order, nu, x, dt, s1, s2, a1, a2, w1_q, w2_q)


# speedup vs baseline: 2.8843x; 2.8843x over previous
import jax
import jax.numpy as jnp
from jax import lax
from jax.experimental import pallas as pl
from jax.experimental.pallas import tpu as pltpu
from jax.experimental.pallas import tpu_sc as plsc

E = 64
TOPK = 2
DMODEL = 1024
DFF = 512
T = 64
FP8_MAX = 448.0
L = 16
NBUF = 3


def _routing_body(gt_hbm, dense_hbm, order_hbm, nu_hbm,
                  gt_v, dense_v, used_v, ord_v, nu_v):
    c = lax.axis_index("c")
    s = lax.axis_index("s")

    @pl.when(jnp.logical_and(c == 0, s == 0))
    def _work():
        pltpu.sync_copy(gt_hbm, gt_v)
        zf = jnp.zeros((L,), jnp.float32)
        zi = jnp.zeros((L,), jnp.int32)

        def _zero_chunk(j, carry):
            for u in range(16):
                dense_v[pl.ds(L * 16 * j + L * u, L)] = zf
            return carry

        lax.fori_loop(0, E * T // (L * 16), _zero_chunk, 0)
        for k in range(E // L):
            used_v[pl.ds(L * k, L)] = zi
            ord_v[pl.ds(L * k, L)] = zi

        ones = jnp.ones((L,), jnp.int32)
        for g in range(T // L):
            def _step(eb, carry):
                m1, i1, m2, i2 = carry
                for u in range(8):
                    e = eb * 8 + u
                    v = gt_v[e, pl.ds(L * g, L)]
                    e_vec = jnp.full((L,), 1, jnp.int32) * e
                    gt1 = v > m1
                    gt2 = jnp.logical_and(jnp.logical_not(gt1), v > m2)
                    m2n = jnp.where(gt1, m1, jnp.where(gt2, v, m2))
                    i2 = jnp.where(gt1, i1, jnp.where(gt2, e_vec, i2))
                    m2 = m2n
                    i1 = jnp.where(gt1, e_vec, i1)
                    m1 = jnp.where(gt1, v, m1)
                return m1, i1, m2, i2

            init = (jnp.full((L,), -jnp.inf, jnp.float32), zi,
                    jnp.full((L,), -jnp.inf, jnp.float32), zi)
            m1, i1, m2, i2 = lax.fori_loop(0, E // 8, _step, init)
            r = jnp.exp(m2 - m1)
            w1v = 1.0 / (1.0 + r)
            w2v = r / (1.0 + r)
            tvec = (lax.iota(jnp.int32, L) + L * g) * E
            plsc.store_scatter(dense_v, [tvec + i1], w1v)
            plsc.store_scatter(dense_v, [tvec + i2], w2v)
            plsc.store_scatter(used_v, [i1], ones)
            plsc.store_scatter(used_v, [i2], ones)

        total = jnp.int32(0)
        for k in range(E // L):
            mk = used_v[pl.ds(L * k, L)] > 0
            inc = jnp.where(mk, 1, 0)
            pos = plsc.cumsum(inc) - 1 + total
            ids = lax.iota(jnp.int32, L) + L * k
            plsc.store_scatter(ord_v, [pos], ids, mask=mk)
            total = total + jnp.sum(inc)
        nu_v[...] = jnp.full((L,), total, jnp.int32)

        pltpu.sync_copy(dense_v, dense_hbm)
        pltpu.sync_copy(ord_v, order_hbm)
        pltpu.sync_copy(nu_v, nu_hbm)


def _routing(gt):
    f = pl.kernel(
        _routing_body,
        out_type=(
            jax.ShapeDtypeStruct((T * E,), jnp.float32),
            jax.ShapeDtypeStruct((E,), jnp.int32),
            jax.ShapeDtypeStruct((L,), jnp.int32),
        ),
        mesh=plsc.VectorSubcoreMesh(core_axis_name="c", subcore_axis_name="s"),
        compiler_params=pltpu.CompilerParams(needs_layout_passes=False),
        scratch_types=[
            pltpu.VMEM((E, T), jnp.float32),
            pltpu.VMEM((T * E,), jnp.float32),
            pltpu.VMEM((E,), jnp.int32),
            pltpu.VMEM((E,), jnp.int32),
            pltpu.VMEM((L,), jnp.int32),
        ],
    )
    return f(gt)


def _moe_body(order_ref, nu_ref, x_ref, dt_ref, s1_ref, s2_ref,
              a1_ref, a2_ref, w1_hbm, w2_hbm, out_ref,
              w1_buf, w2_buf, sem1, sem2):
    nu = nu_ref[0]

    def _issue(i, b):
        e = order_ref[i]
        pltpu.make_async_copy(w1_hbm.at[e], w1_buf.at[b], sem1.at[b]).start()
        pltpu.make_async_copy(w2_hbm.at[e], w2_buf.at[b], sem2.at[b]).start()

    def _wait1(i, b):
        e = order_ref[i]
        pltpu.make_async_copy(w1_hbm.at[e], w1_buf.at[b], sem1.at[b]).wait()

    def _wait2(i, b):
        e = order_ref[i]
        pltpu.make_async_copy(w2_hbm.at[e], w2_buf.at[b], sem2.at[b]).wait()

    for b in range(NBUF):
        @pl.when(b < nu)
        def _prime(b=b):
            _issue(b, b)

    a1 = a1_ref[0]
    a2 = a2_ref[0]
    xq = jnp.clip(x_ref[...] / a1, -FP8_MAX, FP8_MAX)
    dt = dt_ref[...]
    col = lax.broadcasted_iota(jnp.int32, (E, 1), 0)
    out_ref[...] = jnp.zeros_like(out_ref)

    def _super(i_sup, acc):
        base = i_sup * NBUF
        for b in range(NBUF):
            i = base + b

            @pl.when(i < nu)
            def _one(i=i, b=b):
                e = order_ref[i]
                _wait1(i, b)
                h = lax.dot_general(xq, w1_buf[b], (((1,), (1,)), ((), ())),
                                    preferred_element_type=jnp.float32)
                h = h * (a1 * s1_ref[e])
                gate = h[:, :DFF]
                up = h[:, DFF:]
                act = gate * jax.nn.sigmoid(gate) * up
                aq = jnp.clip(act / a2, -FP8_MAX, FP8_MAX)
                _wait2(i, b)
                oe = lax.dot_general(aq, w2_buf[b], (((1,), (1,)), ((), ())),
                                     preferred_element_type=jnp.float32)
                oe = oe * (a2 * s2_ref[e])
                onehot = jnp.where(col == e, 1.0, 0.0)
                wt_col = lax.dot_general(dt, onehot, (((1,), (0,)), ((), ())),
                                         preferred_element_type=jnp.float32)
                out_ref[...] += wt_col * oe

                j = i + NBUF

                @pl.when(j < nu)
                def _next():
                    _issue(j, b)

        return acc

    n_sup = lax.div(nu + (NBUF - 1), NBUF)
    lax.fori_loop(0, n_sup, _super, 0)


def kernel(x, gating_output, w1_q, w2_q, w1_scale, w2_scale, a1_scale,
           a2_scale):
    dense, order, nu = _routing(gating_output.T)
    dt = dense.reshape(T, E)
    s1 = w1_scale.reshape(E)
    s2 = w2_scale.reshape(E)
    a1 = a1_scale.reshape(1)
    a2 = a2_scale.reshape(1)
    return pl.pallas_call(
        _moe_body,
        in_specs=[
            pl.BlockSpec(memory_space=pltpu.SMEM),
            pl.BlockSpec(memory_space=pltpu.SMEM),
            pl.BlockSpec(memory_space=pltpu.VMEM),
            pl.BlockSpec(memory_space=pltpu.VMEM),
            pl.BlockSpec(memory_space=pltpu.SMEM),
            pl.BlockSpec(memory_space=pltpu.SMEM),
            pl.BlockSpec(memory_space=pltpu.SMEM),
            pl.BlockSpec(memory_space=pltpu.SMEM),
            pl.BlockSpec(memory_space=pltpu.MemorySpace.HBM),
            pl.BlockSpec(memory_space=pltpu.MemorySpace.HBM),
        ],
        out_specs=pl.BlockSpec(memory_space=pltpu.VMEM),
        out_shape=jax.ShapeDtypeStruct((T, DMODEL), jnp.float32),
        scratch_shapes=[
            pltpu.VMEM((NBUF, 2 * DFF, DMODEL), jnp.float32),
            pltpu.VMEM((NBUF, DMODEL, DFF), jnp.float32),
            pltpu.SemaphoreType.DMA((NBUF,)),
            pltpu.SemaphoreType.DMA((NBUF,)),
        ],
        compiler_params=pltpu.CompilerParams(
            vmem_limit_bytes=100 * 1024 * 1024),
    )(order, nu, x, dt, s1, s2, a1, a2, w1_q, w2_q)
